# baseline (speedup 1.0000x reference)
import functools

import jax
import jax.numpy as jnp
from jax import lax
from jax.experimental import pallas as pl
from jax.experimental.pallas import tpu as pltpu
from jax.experimental.pallas import tpu_sc as plsc

_SMOOTH = 100.0
_N = 64 * 512 * 512
_NC = 2
_NS = 16
_NW = _NC * _NS
_PW = _N // _NW
_CH = 16384
_NCH = _PW // _CH
_L = 16
_VPC = _CH // _L


def _sc_body(yt_hbm, yp_hbm, out_hbm, yt_buf, yp_buf, res_buf):
    wid = lax.axis_index("s") * _NC + lax.axis_index("c")
    base = wid * _PW

    def chunk_body(c, accs):
        pltpu.sync_copy(yt_hbm.at[pl.ds(base + c * _CH, _CH)], yt_buf)
        pltpu.sync_copy(yp_hbm.at[pl.ds(base + c * _CH, _CH)], yp_buf)

        def vec_body(i, accs):
            acc_i, acc_s = accs
            a = jnp.abs(yt_buf[pl.ds(i * _L, _L)])
            b = jnp.abs(yp_buf[pl.ds(i * _L, _L)])
            m = yt_buf[pl.ds(i * _L, _L)] != jnp.float32(0.85)
            a = jnp.where(m, a, jnp.float32(0.0))
            b = jnp.where(m, b, jnp.float32(0.0))
            return acc_i + a * b, acc_s + (a + b)

        return lax.fori_loop(0, _VPC, vec_body, accs)

    zeros = jnp.zeros((_L,), jnp.float32)
    acc_i, acc_s = lax.fori_loop(0, _NCH, chunk_body, (zeros, zeros))
    res_buf[pl.ds(0, _L)] = acc_i
    res_buf[pl.ds(_L, _L)] = acc_s
    pltpu.sync_copy(res_buf, out_hbm.at[wid])


@jax.jit
def _partials(yt_flat, yp_flat):
    return pl.kernel(
        _sc_body,
        out_type=jax.ShapeDtypeStruct((_NW, 2 * _L), jnp.float32),
        mesh=plsc.VectorSubcoreMesh(core_axis_name="c", subcore_axis_name="s"),
        scratch_types=[
            pltpu.VMEM((_CH,), jnp.float32),
            pltpu.VMEM((_CH,), jnp.float32),
            pltpu.VMEM((2 * _L,), jnp.float32),
        ],
    )(yt_flat, yp_flat)


def kernel(y_true, y_pred):
    batch_size = y_true.shape[0]
    out = _partials(y_true.reshape(_N), y_pred.reshape(_N))
    intersection = out[:, :_L].sum()
    sum_ = out[:, _L:].sum()
    jac = (intersection + _SMOOTH) / (sum_ - intersection + _SMOOTH)
    return (1.0 - jac) * _SMOOTH / batch_size


# speedup vs baseline: 1.4100x; 1.4100x over previous
import jax
import jax.numpy as jnp
from jax import lax
from jax.experimental import pallas as pl
from jax.experimental.pallas import tpu as pltpu
from jax.experimental.pallas import tpu_sc as plsc

_SMOOTH = 100.0
_N = 64 * 512 * 512
_NC = 2
_NS = 16
_NW = _NC * _NS
_PW = _N // _NW
_CH = 16384
_NCH = _PW // _CH
_L = 16
_U = 8
_NACC = 4


def _sc_body(yt_hbm, yp_hbm, out_hbm, yt_buf, yp_buf, res_buf,
             sem_t0, sem_t1, sem_p0, sem_p1):
    wid = lax.axis_index("s") * _NC + lax.axis_index("c")
    base = wid * _PW
    sem_t = (sem_t0, sem_t1)
    sem_p = (sem_p0, sem_p1)

    def start(k, b):
        pltpu.async_copy(yt_hbm.at[pl.ds(base + k * _CH, _CH)],
                         yt_buf.at[b], sem_t[b])
        pltpu.async_copy(yp_hbm.at[pl.ds(base + k * _CH, _CH)],
                         yp_buf.at[b], sem_p[b])

    start(0, 0)
    start(1, 1)

    zeros = jnp.zeros((_L,), jnp.float32)
    init = (zeros,) * (2 * _NACC)

    def outer(g, accs):
        for b in range(2):
            k = 2 * g + b
            pltpu.make_async_copy(yt_hbm.at[pl.ds(base, _CH)],
                                  yt_buf.at[b], sem_t[b]).wait()
            pltpu.make_async_copy(yp_hbm.at[pl.ds(base, _CH)],
                                  yp_buf.at[b], sem_p[b]).wait()

            def vec_body(i, accs, b=b):
                accs = list(accs)
                for u in range(_U):
                    off = i * (_U * _L) + u * _L
                    yt = yt_buf[b, pl.ds(off, _L)]
                    a = jnp.abs(yt)
                    p = jnp.abs(yp_buf[b, pl.ds(off, _L)])
                    m = yt != jnp.float32(0.85)
                    a = jnp.where(m, a, jnp.float32(0.0))
                    p = jnp.where(m, p, jnp.float32(0.0))
                    j = u % _NACC
                    accs[j] = accs[j] + a * p
                    accs[_NACC + j] = accs[_NACC + j] + (a + p)
                return tuple(accs)

            accs = lax.fori_loop(0, _CH // (_U * _L), vec_body, accs)

            @pl.when(k + 2 < _NCH)
            def _(k=k, b=b):
                start(k + 2, b)
        return accs

    accs = lax.fori_loop(0, _NCH // 2, outer, init)
    acc_i = accs[0] + accs[1] + accs[2] + accs[3]
    acc_s = accs[4] + accs[5] + accs[6] + accs[7]
    res_buf[pl.ds(0, _L)] = acc_i
    res_buf[pl.ds(_L, _L)] = acc_s
    pltpu.sync_copy(res_buf, out_hbm.at[wid])


@jax.jit
def _partials(yt_flat, yp_flat):
    return pl.kernel(
        _sc_body,
        out_type=jax.ShapeDtypeStruct((_NW, 2 * _L), jnp.float32),
        mesh=plsc.VectorSubcoreMesh(core_axis_name="c", subcore_axis_name="s"),
        scratch_types=[
            pltpu.VMEM((2, _CH), jnp.float32),
            pltpu.VMEM((2, _CH), jnp.float32),
            pltpu.VMEM((2 * _L,), jnp.float32),
            pltpu.SemaphoreType.DMA,
            pltpu.SemaphoreType.DMA,
            pltpu.SemaphoreType.DMA,
            pltpu.SemaphoreType.DMA,
        ],
    )(yt_flat, yp_flat)


def kernel(y_true, y_pred):
    batch_size = y_true.shape[0]
    out = _partials(y_true.reshape(_N), y_pred.reshape(_N))
    intersection = out[:, :_L].sum()
    sum_ = out[:, _L:].sum()
    jac = (intersection + _SMOOTH) / (sum_ - intersection + _SMOOTH)
    return (1.0 - jac) * _SMOOTH / batch_size


# speedup vs baseline: 2.9940x; 2.1234x over previous
import jax
import jax.numpy as jnp
from jax import lax
from jax.experimental import pallas as pl
from jax.experimental.pallas import tpu as pltpu

_SMOOTH = 100.0
_N = 64 * 512 * 512
_ROWS = _N // 512
_BR = 256
_G = _ROWS // _BR


def _tc_body(yt_ref, yp_ref, oi_ref, os_ref):
    x = yt_ref[...]
    y = yp_ref[...]
    a = jnp.abs(x)
    b = jnp.abs(y)
    m = x != jnp.float32(0.85)
    a = jnp.where(m, a, jnp.float32(0.0))
    b = jnp.where(m, b, jnp.float32(0.0))
    p = a * b
    s = a + b
    pi = p[0:8]
    si = s[0:8]
    for k in range(1, _BR // 8):
        pi = pi + p[8 * k:8 * k + 8]
        si = si + s[8 * k:8 * k + 8]
    oi_ref[...] = pi
    os_ref[...] = si


@jax.jit
def _tc_partials(yt, yp):
    return pl.pallas_call(
        _tc_body,
        grid=(_G,),
        in_specs=[
            pl.BlockSpec((_BR, 512), lambda i: (i, 0)),
            pl.BlockSpec((_BR, 512), lambda i: (i, 0)),
        ],
        out_specs=[
            pl.BlockSpec((8, 512), lambda i: (i, 0)),
            pl.BlockSpec((8, 512), lambda i: (i, 0)),
        ],
        out_shape=[
            jax.ShapeDtypeStruct((_G * 8, 512), jnp.float32),
            jax.ShapeDtypeStruct((_G * 8, 512), jnp.float32),
        ],
        compiler_params=pltpu.CompilerParams(
            dimension_semantics=("arbitrary",),
        ),
    )(yt, yp)


def kernel(y_true, y_pred):
    batch_size = y_true.shape[0]
    oi, os = _tc_partials(y_true.reshape(_ROWS, 512), y_pred.reshape(_ROWS, 512))
    intersection = oi.sum()
    sum_ = os.sum()
    jac = (intersection + _SMOOTH) / (sum_ - intersection + _SMOOTH)
    return (1.0 - jac) * _SMOOTH / batch_size


# speedup vs baseline: 3.1765x; 1.0610x over previous
import jax
import jax.numpy as jnp
from jax import lax
from jax.experimental import pallas as pl
from jax.experimental.pallas import tpu as pltpu

_SMOOTH = 100.0
_N = 64 * 512 * 512
_ROWS = _N // 512
_BR = 256
_G = _ROWS // _BR


def _tc_body(yt_ref, yp_ref, oi_ref, os_ref):
    pi = jnp.zeros((8, 512), jnp.float32)
    si = jnp.zeros((8, 512), jnp.float32)
    for k in range(_BR // 8):
        x = yt_ref[8 * k:8 * k + 8, :]
        y = yp_ref[8 * k:8 * k + 8, :]
        a = jnp.abs(x)
        b = jnp.abs(y)
        m = x != jnp.float32(0.85)
        a = jnp.where(m, a, jnp.float32(0.0))
        b = jnp.where(m, b, jnp.float32(0.0))
        pi = pi + a * b
        si = si + (a + b)
    oi_ref[...] = pi
    os_ref[...] = si


@jax.jit
def _tc_partials(yt, yp):
    return pl.pallas_call(
        _tc_body,
        grid=(_G,),
        in_specs=[
            pl.BlockSpec((_BR, 512), lambda i: (i, 0)),
            pl.BlockSpec((_BR, 512), lambda i: (i, 0)),
        ],
        out_specs=[
            pl.BlockSpec((8, 512), lambda i: (i, 0)),
            pl.BlockSpec((8, 512), lambda i: (i, 0)),
        ],
        out_shape=[
            jax.ShapeDtypeStruct((_G * 8, 512), jnp.float32),
            jax.ShapeDtypeStruct((_G * 8, 512), jnp.float32),
        ],
        compiler_params=pltpu.CompilerParams(
            dimension_semantics=("arbitrary",),
        ),
    )(yt, yp)


def kernel(y_true, y_pred):
    batch_size = y_true.shape[0]
    oi, os = _tc_partials(y_true.reshape(_ROWS, 512), y_pred.reshape(_ROWS, 512))
    intersection = oi.sum()
    sum_ = os.sum()
    jac = (intersection + _SMOOTH) / (sum_ - intersection + _SMOOTH)
    return (1.0 - jac) * _SMOOTH / batch_size
